# baseline (speedup 1.0000x reference)
import functools

import jax
import jax.numpy as jnp
from jax import lax
from jax.experimental import pallas as pl
from jax.experimental.pallas import tpu as pltpu
from jax.experimental.pallas import tpu_sc as plsc

_DIM = 96
_GROUP = 2
_K = 5
_NH = 10
_H = 224
_W = 224
_HW = _H * _W
_NCHUNK = 32
_CHUNK = _HW // _NCHUNK
_VREGS = _CHUNK // 16
_NBUCKET = 1 << _NH
_SPLIT = 4
_SUB = _HW // _SPLIT


_RB = 56
_NR = _H // _RB


def _stats_body(w_ref, x_ref, psum_ref, dec_ref, f_ref):
    xb = x_ref[0]
    wc = w_ref[...]
    pw = jnp.left_shift(
        1, lax.broadcasted_iota(jnp.int32, (_NH, 1), 0)).astype(jnp.float32)
    f_ref[0, 0, :, :, _NH:] = jnp.zeros((_RB, _W, 16 - _NH), jnp.float32)
    for r in range(_RB):
        proj = lax.dot_general(wc, xb[:, r, :], (((1,), (0,)), ((), ())),
                               preferred_element_type=jnp.float32)
        dec = jnp.sum(jnp.where(proj[0:_NH] > 0, pw, 0.0), axis=0)
        dec_ref[0, 0, r, :] = dec
        f_ref[0, 0, r, :, 0:_NH] = jnp.transpose(proj[_NH:2 * _NH])
    psum_ref[...] = jnp.sum(xb * xb, axis=(0, 1)).reshape(1, 1, 1, _W)


def _stats_call(x4, wc):
    return pl.pallas_call(
        _stats_body,
        grid=(2, _NR),
        in_specs=[
            pl.BlockSpec((2 * _NH, _DIM), lambda n, j: (0, 0)),
            pl.BlockSpec((1, _DIM, _RB, _W), lambda n, j: (n, 0, j, 0)),
        ],
        out_specs=[
            pl.BlockSpec((1, 1, 1, _W), lambda n, j: (n, j, 0, 0)),
            pl.BlockSpec((1, 1, _RB, _W), lambda n, j: (n, j, 0, 0)),
            pl.BlockSpec((1, 1, _RB, _W, 16), lambda n, j: (n, j, 0, 0, 0)),
        ],
        out_shape=[
            jax.ShapeDtypeStruct((2, _NR, 1, _W), jnp.float32),
            jax.ShapeDtypeStruct((2, _NR, _RB, _W), jnp.float32),
            jax.ShapeDtypeStruct((2, _NR, _RB, _W, 16), jnp.float32),
        ],
    )(wc, x4)


def _dup_stats(d, iota, tmp16):
    tmp16[...] = d
    within = jnp.zeros((16,), jnp.int32)
    later = jnp.zeros((16,), jnp.int32)
    one = jnp.ones((16,), jnp.int32)
    zero = jnp.zeros((16,), jnp.int32)
    for j in range(16):
        dj = plsc.load_gather(tmp16, [jnp.full((16,), j, jnp.int32)])
        eq = d == dj
        within = within + jnp.where(jnp.logical_and(eq, iota > j), one, zero)
        later = later + jnp.where(jnp.logical_and(eq, iota < j), one, zero)
    return within, later == 0


def _csum16(c, iota, tmp16):
    for sh in (1, 2, 4, 8):
        tmp16[...] = c
        shifted = plsc.load_gather(tmp16, [jnp.maximum(iota - sh, 0)])
        c = c + jnp.where(iota >= sh, shifted, jnp.zeros((16,), jnp.int32))
    return c


def _sc_sort_body(dec_hbm, ft_hbm, fs_hbm,
                  dec_v, hist, all_h, offs, pos_a, pos_b, ft_v, tmp16, sem,
                  shared):
    cid = lax.axis_index("c")
    sid = lax.axis_index("s")
    iota = lax.iota(jnp.int32, 16)
    ones = jnp.ones((16,), jnp.int32)

    def _hist_chunk(t, _):
        w = sid + 16 * t

        def _zero(j, _):
            hist[pl.ds(j * 16, 16)] = jnp.zeros((16,), jnp.int32)
            return 0
        lax.fori_loop(0, _NBUCKET // 16, _zero, 0)
        pltpu.sync_copy(dec_hbm.at[pl.ds(w * _CHUNK, _CHUNK)], dec_v)

        def _acc(i, _):
            d = dec_v[pl.ds(i * 16, 16)]
            r, is_end = _dup_stats(d, iota, tmp16)
            plsc.addupdate_scatter(hist, [d], r + 1, mask=is_end)
            return 0
        lax.fori_loop(0, _VREGS, _acc, 0)
        pltpu.sync_copy(hist, shared.at[w])
        return 0
    lax.fori_loop(0, 2, _hist_chunk, 0)
    plsc.subcore_barrier()

    g = cid * 16 + sid
    pltpu.sync_copy(shared, all_h)

    def _offs(j, carry):
        sl = pl.ds(j * 16, 16)

        def _sum(w, tp):
            tot, pre = tp
            hrow = all_h[w, sl]
            return (tot + hrow,
                    pre + jnp.where(w < g, hrow, jnp.zeros((16,), jnp.int32)))
        tot, pre = lax.fori_loop(
            0, _NCHUNK, _sum,
            (jnp.zeros((16,), jnp.int32), jnp.zeros((16,), jnp.int32)))
        cs = _csum16(tot, iota, tmp16)
        offs[sl] = carry + cs - tot + pre
        tmp16[...] = cs
        tsplat = plsc.load_gather(tmp16, [jnp.full((16,), 15, jnp.int32)])
        return carry + tsplat
    lax.fori_loop(0, _NBUCKET // 16, _offs, jnp.zeros((16,), jnp.int32))

    pltpu.sync_copy(dec_hbm.at[pl.ds(g * _CHUNK, _CHUNK)], dec_v)

    def _rank(i, _):
        d = dec_v[pl.ds(i * 16, 16)]
        cur = plsc.load_gather(offs, [d])
        r, is_end = _dup_stats(d, iota, tmp16)
        pos = cur + r
        row = i // 8
        col = (i % 8) * 16 + iota
        plsc.store_scatter(pos_a, [row * ones, col], pos)
        plsc.addupdate_scatter(offs, [d], r + 1, mask=is_end)
        return 0
    lax.fori_loop(0, 96, _rank, 0)
    for i in (96, 97):
        d = dec_v[pl.ds(i * 16, 16)]
        cur = plsc.load_gather(offs, [d])
        r, is_end = _dup_stats(d, iota, tmp16)
        plsc.store_scatter(pos_b, [(i - 96) * ones, iota], cur + r)
        plsc.addupdate_scatter(offs, [d], r + 1, mask=is_end)

    pltpu.sync_copy(ft_hbm.at[pl.ds(g * _CHUNK, _CHUNK)], ft_v)
    copies = []
    for i in range(12):
        copies.append(pltpu.async_copy(
            ft_v.at[pl.ds(i * 128, 128)], fs_hbm.at[pos_a.at[i]], sem))
    for t in range(2):
        copies.append(pltpu.async_copy(
            ft_v.at[pl.ds(1536 + t * 16, 16)], fs_hbm.at[pos_b.at[t]], sem))
    for cp in copies:
        cp.wait()


_sc_sort_cache = []


def _sc_sort(dec_b, ft):
    if not _sc_sort_cache:
        _sc_sort_cache.append(functools.partial(
            pl.kernel,
            out_type=jax.ShapeDtypeStruct((_HW, 16), jnp.float32),
            mesh=plsc.VectorSubcoreMesh(
                core_axis_name="c", subcore_axis_name="s"),
            compiler_params=pltpu.CompilerParams(
                needs_layout_passes=False, use_tc_tiling_on_sc=False),
            scratch_types=[
                pltpu.VMEM((_CHUNK,), jnp.int32),
                pltpu.VMEM((_NBUCKET,), jnp.int32),
                pltpu.VMEM((_NCHUNK, _NBUCKET), jnp.int32),
                pltpu.VMEM((_NBUCKET,), jnp.int32),
                pltpu.VMEM((12, 128), jnp.int32),
                pltpu.VMEM((2, 16), jnp.int32),
                pltpu.VMEM((_CHUNK, 16), jnp.float32),
                pltpu.VMEM((16,), jnp.int32),
                pltpu.SemaphoreType.DMA,
                pltpu.VMEM_SHARED((_NCHUNK, _NBUCKET), jnp.int32),
            ],
        )(_sc_sort_body))
    return _sc_sort_cache[0](dec_b, ft)


def _pass_a_body(x_ref, sf_ref, o_ref):
    xb = x_ref[0]
    sg = jax.nn.sigmoid(sf_ref[0])
    xp = jnp.concatenate(
        [xb[:, :, 2:3], xb[:, :, 1:2], xb,
         xb[:, :, _W - 2:_W - 1], xb[:, :, _W - 3:_W - 2]], axis=2)
    acc = xp[:, :, 0:_W] * sg[0][None]
    for k in range(1, _K):
        acc = acc + xp[:, :, k:k + _W] * sg[k][None]
    o_ref[0] = acc


def _pass_a_call(x4, sf):
    return pl.pallas_call(
        _pass_a_body,
        grid=(2, 6),
        in_specs=[
            pl.BlockSpec((1, 16, _H, _W), lambda n, cc: (n, cc, 0, 0)),
            pl.BlockSpec((1, _K, _H, _W), lambda n, cc: (cc // 3, 0, 0, 0)),
        ],
        out_specs=pl.BlockSpec((1, 16, _H, _W), lambda n, cc: (n, cc, 0, 0)),
        out_shape=jax.ShapeDtypeStruct((2, _DIM, _H, _W), jnp.float32),
    )(x4, sf)


def _pass_b_body(x_ref, sf_ref, xo_ref, g_ref, b_ref, o_ref):
    xb = x_ref[0]
    sg = jax.nn.sigmoid(sf_ref[0])
    hp = jnp.concatenate(
        [xb[:, 2:3, :], xb[:, 1:2, :], xb,
         xb[:, _H - 2:_H - 1, :], xb[:, _H - 3:_H - 2, :]], axis=1)
    acc = hp[:, 0:_H, :] * sg[0][None]
    for k in range(1, _K):
        acc = acc + hp[:, k:k + _H, :] * sg[k][None]
    gb = g_ref[:, 0:1].reshape(16, 1, 1)
    bb = b_ref[:, 0:1].reshape(16, 1, 1)
    o_ref[0] = gb * acc + bb * xo_ref[0]


def _pass_b_call(out1, sf, x4, gam, bet):
    return pl.pallas_call(
        _pass_b_body,
        grid=(2, 6),
        in_specs=[
            pl.BlockSpec((1, 16, _H, _W), lambda n, cc: (n, cc, 0, 0)),
            pl.BlockSpec((1, _K, _H, _W), lambda n, cc: (cc // 3, 0, 0, 0)),
            pl.BlockSpec((1, 16, _H, _W), lambda n, cc: (n, cc, 0, 0)),
            pl.BlockSpec((16, 128), lambda n, cc: (cc, 0)),
            pl.BlockSpec((16, 128), lambda n, cc: (cc, 0)),
        ],
        out_specs=pl.BlockSpec((1, 16, _H, _W), lambda n, cc: (n, cc, 0, 0)),
        out_shape=jax.ShapeDtypeStruct((2, _DIM, _H, _W), jnp.float32),
    )(out1, sf, x4, gam, bet)


def _strip_pass(img4, wc, pass_call, *extra):
    psum, dec, f = _stats_call(img4, wc)
    best = jnp.argmax(jnp.sum(psum, axis=(1, 2, 3)))
    decr = dec.reshape(2, _HW)
    dec_b = jnp.where(best == 0, decr[0], decr[1]).astype(jnp.int32)
    f_b = jnp.where(best == 0, f[0], f[1])
    ft = f_b.reshape(_HW, 16)
    fs = _sc_sort(dec_b, ft)
    sf = jnp.transpose(fs[:, :_NH]).reshape(_GROUP, _K, _H, _W)
    return pass_call(img4, sf, *extra)


def kernel(x, W_conv_H, W_conv_W, rv_H, rv_W, gamma, beta):
    wc1 = jnp.concatenate([rv_H, W_conv_H], axis=0)
    wc2 = jnp.concatenate([rv_W, W_conv_W], axis=0)
    out1 = _strip_pass(x, wc1, _pass_a_call)
    gam = jnp.broadcast_to(gamma.reshape(_DIM, 1), (_DIM, 128))
    bet = jnp.broadcast_to(beta.reshape(_DIM, 1), (_DIM, 128))
    return _strip_pass(out1, wc2, _pass_b_call, x, gam, bet)


# speedup vs baseline: 1.0855x; 1.0855x over previous
import functools

import jax
import jax.numpy as jnp
from jax import lax
from jax.experimental import pallas as pl
from jax.experimental.pallas import tpu as pltpu
from jax.experimental.pallas import tpu_sc as plsc

_DIM = 96
_GROUP = 2
_K = 5
_NH = 10
_H = 224
_W = 224
_HW = _H * _W
_NCHUNK = 32
_CHUNK = _HW // _NCHUNK
_VREGS = _CHUNK // 16
_NBUCKET = 1 << _NH
_SPLIT = 4
_SUB = _HW // _SPLIT


_RB = 56
_NR = _H // _RB


def _stats_body(w_ref, x_ref, psum_ref, dec_ref, f_ref):
    xb = x_ref[0]
    wc = w_ref[...]
    pw = jnp.left_shift(
        1, lax.broadcasted_iota(jnp.int32, (_NH, 1), 0)).astype(jnp.float32)
    f_ref[0, 0, _NH:, :, :] = jnp.zeros((16 - _NH, _RB, _W), jnp.float32)
    for r in range(_RB):
        proj = lax.dot_general(wc, xb[:, r, :], (((1,), (0,)), ((), ())),
                               preferred_element_type=jnp.float32)
        dec = jnp.sum(jnp.where(proj[0:_NH] > 0, pw, 0.0), axis=0)
        dec_ref[0, 0, r, :] = dec
        f_ref[0, 0, 0:_NH, r, :] = proj[_NH:2 * _NH]
    psum_ref[...] = jnp.sum(xb * xb, axis=(0, 1)).reshape(1, 1, 1, _W)


def _stats_call(x4, wc):
    return pl.pallas_call(
        _stats_body,
        grid=(2, _NR),
        in_specs=[
            pl.BlockSpec((2 * _NH, _DIM), lambda n, j: (0, 0)),
            pl.BlockSpec((1, _DIM, _RB, _W), lambda n, j: (n, 0, j, 0)),
        ],
        out_specs=[
            pl.BlockSpec((1, 1, 1, _W), lambda n, j: (n, j, 0, 0)),
            pl.BlockSpec((1, 1, _RB, _W), lambda n, j: (n, j, 0, 0)),
            pl.BlockSpec((1, 1, 16, _RB, _W), lambda n, j: (n, j, 0, 0, 0)),
        ],
        out_shape=[
            jax.ShapeDtypeStruct((2, _NR, 1, _W), jnp.float32),
            jax.ShapeDtypeStruct((2, _NR, _RB, _W), jnp.float32),
            jax.ShapeDtypeStruct((2, _NR, 16, _RB, _W), jnp.float32),
        ],
    )(wc, x4)


def _dup_stats(d, iota, tmp16):
    tmp16[...] = d
    within = jnp.zeros((16,), jnp.int32)
    later = jnp.zeros((16,), jnp.int32)
    one = jnp.ones((16,), jnp.int32)
    zero = jnp.zeros((16,), jnp.int32)
    for j in range(16):
        dj = plsc.load_gather(tmp16, [jnp.full((16,), j, jnp.int32)])
        eq = d == dj
        within = within + jnp.where(jnp.logical_and(eq, iota > j), one, zero)
        later = later + jnp.where(jnp.logical_and(eq, iota < j), one, zero)
    return within, later == 0


def _csum16(c, iota, tmp16):
    for sh in (1, 2, 4, 8):
        tmp16[...] = c
        shifted = plsc.load_gather(tmp16, [jnp.maximum(iota - sh, 0)])
        c = c + jnp.where(iota >= sh, shifted, jnp.zeros((16,), jnp.int32))
    return c


def _sc_sort_body(dec_hbm, ft_hbm, fs_hbm,
                  dec_v, hist, hist16, all_h, offs, pos_a, pos_b, ft_v, tmp16,
                  sem, shared):
    cid = lax.axis_index("c")
    sid = lax.axis_index("s")
    iota = lax.iota(jnp.int32, 16)
    ones = jnp.ones((16,), jnp.int32)

    def _hist_chunk(t, _):
        w = sid + 16 * t

        def _zero(j, _):
            for u in range(8):
                hist16[pl.ds((j * 8 + u) * 16, 16)] = (
                    jnp.zeros((16,), jnp.int32))
            return 0
        lax.fori_loop(0, 16 * _NBUCKET // 128, _zero, 0)
        pltpu.sync_copy(dec_hbm.at[pl.ds(w * _CHUNK, _CHUNK)], dec_v)

        def _acc(i, _):
            d = dec_v[pl.ds(i * 16, 16)]
            plsc.addupdate_scatter(hist16, [iota * _NBUCKET + d], ones)
            return 0
        lax.fori_loop(0, _VREGS, _acc, 0)

        def _red(j, _):
            acc = jnp.zeros((16,), jnp.int32)
            for l in range(16):
                acc = acc + hist16[pl.ds(l * _NBUCKET + j * 16, 16)]
            hist[pl.ds(j * 16, 16)] = acc
            return 0
        lax.fori_loop(0, _NBUCKET // 16, _red, 0)
        pltpu.sync_copy(hist, shared.at[w])
        return 0
    lax.fori_loop(0, 2, _hist_chunk, 0)
    plsc.subcore_barrier()

    g = cid * 16 + sid
    pltpu.sync_copy(shared, all_h)

    def _offs(j, carry):
        sl = pl.ds(j * 16, 16)

        def _sum(w, tp):
            tot, pre = tp
            hrow = all_h[w, sl]
            return (tot + hrow,
                    pre + jnp.where(w < g, hrow, jnp.zeros((16,), jnp.int32)))
        tot, pre = lax.fori_loop(
            0, _NCHUNK, _sum,
            (jnp.zeros((16,), jnp.int32), jnp.zeros((16,), jnp.int32)))
        cs = _csum16(tot, iota, tmp16)
        offs[sl] = carry + cs - tot + pre
        tmp16[...] = cs
        tsplat = plsc.load_gather(tmp16, [jnp.full((16,), 15, jnp.int32)])
        return carry + tsplat
    lax.fori_loop(0, _NBUCKET // 16, _offs, jnp.zeros((16,), jnp.int32))

    pltpu.sync_copy(dec_hbm.at[pl.ds(g * _CHUNK, _CHUNK)], dec_v)

    def _rank(i, _):
        d = dec_v[pl.ds(i * 16, 16)]
        cur = plsc.load_gather(offs, [d])
        r, is_end = _dup_stats(d, iota, tmp16)
        pos = cur + r
        row = i // 8
        col = (i % 8) * 16 + iota
        plsc.store_scatter(pos_a, [row * ones, col], pos)
        plsc.addupdate_scatter(offs, [d], r + 1, mask=is_end)
        return 0
    lax.fori_loop(0, 96, _rank, 0)
    for i in (96, 97):
        d = dec_v[pl.ds(i * 16, 16)]
        cur = plsc.load_gather(offs, [d])
        r, is_end = _dup_stats(d, iota, tmp16)
        plsc.store_scatter(pos_b, [(i - 96) * ones, iota], cur + r)
        plsc.addupdate_scatter(offs, [d], r + 1, mask=is_end)

    pltpu.sync_copy(ft_hbm.at[pl.ds(g * _CHUNK, _CHUNK)], ft_v)
    copies = []
    for i in range(12):
        copies.append(pltpu.async_copy(
            ft_v.at[pl.ds(i * 128, 128)], fs_hbm.at[pos_a.at[i]], sem))
    for t in range(2):
        copies.append(pltpu.async_copy(
            ft_v.at[pl.ds(1536 + t * 16, 16)], fs_hbm.at[pos_b.at[t]], sem))
    for cp in copies:
        cp.wait()


_sc_sort_cache = []


def _sc_sort(dec_b, ft):
    if not _sc_sort_cache:
        _sc_sort_cache.append(functools.partial(
            pl.kernel,
            out_type=jax.ShapeDtypeStruct((_HW, 16), jnp.float32),
            mesh=plsc.VectorSubcoreMesh(
                core_axis_name="c", subcore_axis_name="s"),
            compiler_params=pltpu.CompilerParams(
                needs_layout_passes=False, use_tc_tiling_on_sc=False),
            scratch_types=[
                pltpu.VMEM((_CHUNK,), jnp.int32),
                pltpu.VMEM((_NBUCKET,), jnp.int32),
                pltpu.VMEM((16 * _NBUCKET,), jnp.int32),
                pltpu.VMEM((_NCHUNK, _NBUCKET), jnp.int32),
                pltpu.VMEM((_NBUCKET,), jnp.int32),
                pltpu.VMEM((12, 128), jnp.int32),
                pltpu.VMEM((2, 16), jnp.int32),
                pltpu.VMEM((_CHUNK, 16), jnp.float32),
                pltpu.VMEM((16,), jnp.int32),
                pltpu.SemaphoreType.DMA,
                pltpu.VMEM_SHARED((_NCHUNK, _NBUCKET), jnp.int32),
            ],
        )(_sc_sort_body))
    return _sc_sort_cache[0](dec_b, ft)


def _pass_a_body(x_ref, sf_ref, o_ref):
    xb = x_ref[0]
    sg = jax.nn.sigmoid(sf_ref[0])
    xp = jnp.concatenate(
        [xb[:, :, 2:3], xb[:, :, 1:2], xb,
         xb[:, :, _W - 2:_W - 1], xb[:, :, _W - 3:_W - 2]], axis=2)
    acc = xp[:, :, 0:_W] * sg[0][None]
    for k in range(1, _K):
        acc = acc + xp[:, :, k:k + _W] * sg[k][None]
    o_ref[0] = acc


def _pass_a_call(x4, sf):
    return pl.pallas_call(
        _pass_a_body,
        grid=(2, 6),
        in_specs=[
            pl.BlockSpec((1, 16, _H, _W), lambda n, cc: (n, cc, 0, 0)),
            pl.BlockSpec((1, _K, _H, _W), lambda n, cc: (cc // 3, 0, 0, 0)),
        ],
        out_specs=pl.BlockSpec((1, 16, _H, _W), lambda n, cc: (n, cc, 0, 0)),
        out_shape=jax.ShapeDtypeStruct((2, _DIM, _H, _W), jnp.float32),
    )(x4, sf)


def _pass_b_body(x_ref, sf_ref, xo_ref, g_ref, b_ref, o_ref):
    xb = x_ref[0]
    sg = jax.nn.sigmoid(sf_ref[0])
    hp = jnp.concatenate(
        [xb[:, 2:3, :], xb[:, 1:2, :], xb,
         xb[:, _H - 2:_H - 1, :], xb[:, _H - 3:_H - 2, :]], axis=1)
    acc = hp[:, 0:_H, :] * sg[0][None]
    for k in range(1, _K):
        acc = acc + hp[:, k:k + _H, :] * sg[k][None]
    gb = g_ref[:, 0:1].reshape(16, 1, 1)
    bb = b_ref[:, 0:1].reshape(16, 1, 1)
    o_ref[0] = gb * acc + bb * xo_ref[0]


def _pass_b_call(out1, sf, x4, gam, bet):
    return pl.pallas_call(
        _pass_b_body,
        grid=(2, 6),
        in_specs=[
            pl.BlockSpec((1, 16, _H, _W), lambda n, cc: (n, cc, 0, 0)),
            pl.BlockSpec((1, _K, _H, _W), lambda n, cc: (cc // 3, 0, 0, 0)),
            pl.BlockSpec((1, 16, _H, _W), lambda n, cc: (n, cc, 0, 0)),
            pl.BlockSpec((16, 128), lambda n, cc: (cc, 0)),
            pl.BlockSpec((16, 128), lambda n, cc: (cc, 0)),
        ],
        out_specs=pl.BlockSpec((1, 16, _H, _W), lambda n, cc: (n, cc, 0, 0)),
        out_shape=jax.ShapeDtypeStruct((2, _DIM, _H, _W), jnp.float32),
    )(out1, sf, x4, gam, bet)


def _strip_pass(img4, wc, pass_call, *extra):
    psum, dec, f = _stats_call(img4, wc)
    best = jnp.argmax(jnp.sum(psum, axis=(1, 2, 3)))
    decr = dec.reshape(2, _HW)
    dec_b = jnp.where(best == 0, decr[0], decr[1]).astype(jnp.int32)
    f_b = jnp.where(best == 0, f[0], f[1])
    ft = jnp.transpose(f_b, (0, 2, 3, 1)).reshape(_HW, 16)
    fs = _sc_sort(dec_b, ft)
    sf = jnp.transpose(fs[:, :_NH]).reshape(_GROUP, _K, _H, _W)
    return pass_call(img4, sf, *extra)


def kernel(x, W_conv_H, W_conv_W, rv_H, rv_W, gamma, beta):
    wc1 = jnp.concatenate([rv_H, W_conv_H], axis=0)
    wc2 = jnp.concatenate([rv_W, W_conv_W], axis=0)
    out1 = _strip_pass(x, wc1, _pass_a_call)
    gam = jnp.broadcast_to(gamma.reshape(_DIM, 1), (_DIM, 128))
    bet = jnp.broadcast_to(beta.reshape(_DIM, 1), (_DIM, 128))
    return _strip_pass(out1, wc2, _pass_b_call, x, gam, bet)
